# baseline (speedup 1.0000x reference)
import jax
import jax.numpy as jnp
from jax.experimental import pallas as pl

_K = 20
_EPS = 1e-5


def _knn_idx(x, k):
    inner = -2.0 * jnp.einsum('bcn,bcm->bnm', x, x)
    xx = jnp.sum(x * x, axis=1)
    dist = xx[:, :, None] + xx[:, None, :] + inner
    _, idx = jax.lax.top_k(-dist, k)
    return idx


def _graph_feature(x, k):
    Bq, C, Nq = x.shape
    idx = _knn_idx(x, k)
    xt = jnp.transpose(x, (0, 2, 1))
    feat = jax.vmap(lambda a, i: a[i])(xt, idx)
    xe = jnp.broadcast_to(xt[:, :, None, :], (Bq, Nq, k, C))
    f = jnp.concatenate([feat - xe, xe], axis=3)
    return jnp.transpose(f, (0, 3, 1, 2))


def _conv2_bn_relu(x, W, g, b):
    y = jnp.einsum('oc,bcnk->bonk', W, x)
    m = jnp.mean(y, axis=(0, 2, 3), keepdims=True)
    v = jnp.var(y, axis=(0, 2, 3), keepdims=True)
    y = (y - m) / jnp.sqrt(v + _EPS)
    y = y * g[None, :, None, None] + b[None, :, None, None]
    return jax.nn.relu(y)


def _conv1_bn_relu(x, W, g, b):
    y = jnp.einsum('oc,bcn->bon', W, x)
    m = jnp.mean(y, axis=(0, 2), keepdims=True)
    v = jnp.var(y, axis=(0, 2), keepdims=True)
    y = (y - m) / jnp.sqrt(v + _EPS)
    y = y * g[None, :, None] + b[None, :, None]
    return jax.nn.relu(y)


def _proj_kernel(xf_ref, w_ref, b_ref, o_ref):
    y = jnp.dot(w_ref[...], xf_ref[0], preferred_element_type=jnp.float32)
    o_ref[0] = y + b_ref[...][:, 0][:, None]


def _proj(xf, W8, b8):
    B, C, N = xf.shape
    O = W8.shape[0]
    Op = 16
    W8p = jnp.zeros((Op, C), jnp.float32).at[:O].set(W8)
    b8p = jnp.zeros((Op, 1), jnp.float32).at[:O, 0].set(b8)
    out = pl.pallas_call(
        _proj_kernel,
        grid=(B,),
        in_specs=[
            pl.BlockSpec((1, C, N), lambda i: (i, 0, 0)),
            pl.BlockSpec((Op, C), lambda i: (0, 0)),
            pl.BlockSpec((Op, 1), lambda i: (0, 0)),
        ],
        out_specs=pl.BlockSpec((1, Op, N), lambda i: (i, 0, 0)),
        out_shape=jax.ShapeDtypeStruct((B, Op, N), jnp.float32),
    )(xf, W8p, b8p)
    return out[:, :O, :]


def kernel(x, W1, g1, b1, W2, g2, b2, W3, g3, b3, W4, g4, b4, W5, g5, b5,
           W6, g6, b6, W7, g7, b7, W8, b8):
    N = x.shape[1]
    xp = jnp.transpose(x, (0, 2, 1))
    x1 = jnp.max(_conv2_bn_relu(_graph_feature(xp, _K), W1, g1, b1), axis=-1)
    x2 = jnp.max(_conv2_bn_relu(_graph_feature(x1, _K), W2, g2, b2), axis=-1)
    x3 = jnp.max(_conv2_bn_relu(_graph_feature(x2, _K), W3, g3, b3), axis=-1)
    x4 = jnp.max(_conv2_bn_relu(_graph_feature(x3, _K), W4, g4, b4), axis=-1)
    xc = jnp.concatenate([x1, x2, x3, x4], axis=1)
    xg = _conv1_bn_relu(xc, W5, g5, b5)
    xm = jnp.max(xg, axis=-1, keepdims=True)
    xe = jnp.broadcast_to(xm, (xm.shape[0], xm.shape[1], N))
    xf = jnp.concatenate([xc, xe], axis=1)
    xf = _conv1_bn_relu(xf, W6, g6, b6)
    xf = _conv1_bn_relu(xf, W7, g7, b7)
    y = _proj(xf, W8, b8)
    return jnp.transpose(y, (0, 2, 1))


# speedup vs baseline: 3.6452x; 3.6452x over previous
import functools

import jax
import jax.numpy as jnp
from jax import lax
from jax.experimental import pallas as pl
from jax.experimental.pallas import tpu as pltpu
from jax.experimental.pallas import tpu_sc as plsc

_K = 20
_EPS = 1e-5
_N = 1024
_B = 4
_P = _B * _N
_RB = 256
_RD = 64


def _knn_body(xblk_ref, xall_ref, xx_ref, idx_ref):
    b = pl.program_id(0)
    i = pl.program_id(1)
    xb = xblk_ref[0]
    xa = xall_ref[0]
    inner = lax.dot_general(xb, xa, (((1,), (1,)), ((), ())),
                            preferred_element_type=jnp.float32)
    xxa = xx_ref[0, 0, :][None, :]
    xxb = xx_ref[0, 0, pl.ds(i * _RB, _RB)].reshape(_RB, 1)
    d = (xxb + xxa) + (-2.0) * inner

    iota = lax.broadcasted_iota(jnp.int32, (_RB, _N), 1)
    cols = []
    for _ in range(_K):
        mn = jnp.min(d, axis=1, keepdims=True)
        am = jnp.min(jnp.where(d == mn, iota, _N), axis=1, keepdims=True)
        cols.append(am)
        d = jnp.where(iota == am, jnp.inf, d)
    idx_ref[0] = jnp.concatenate(cols, axis=1) + b * _N


def _knn(x, xx):
    Bq, Nq, C = x.shape
    return pl.pallas_call(
        _knn_body,
        grid=(Bq, Nq // _RB),
        in_specs=[
            pl.BlockSpec((1, _RB, C), lambda b, i: (b, i, 0)),
            pl.BlockSpec((1, Nq, C), lambda b, i: (b, 0, 0)),
            pl.BlockSpec((1, 1, Nq), lambda b, i: (b, 0, 0)),
        ],
        out_specs=pl.BlockSpec((1, _RB, _K), lambda b, i: (b, i, 0)),
        out_shape=jax.ShapeDtypeStruct((Bq, Nq, _K), jnp.int32),
    )(x, x, xx)


@functools.cache
def _sc_gather(tw):
    info = plsc.get_sparse_core_info()
    NC, NS = info.num_cores, info.num_subcores
    NW = NC * NS
    PW = _P // NW
    CH = 8
    n_chunks = PW // CH
    mesh = plsc.VectorSubcoreMesh(core_axis_name="c", subcore_axis_name="s")

    @functools.partial(
        pl.kernel, mesh=mesh,
        out_type=jax.ShapeDtypeStruct((_P * _K, tw), jnp.float32),
        scratch_types=[
            pltpu.VMEM((CH * _K,), jnp.int32),
            pltpu.VMEM((CH * _K, tw), jnp.float32),
            pltpu.SemaphoreType.DMA,
        ],
    )
    def gather(idx_hbm, tab_hbm, out_hbm, idx_v, rows_v, sem):
        wid = lax.axis_index("s") * NC + lax.axis_index("c")
        base_pt = wid * PW

        def chunk_body(ci, carry):
            r0 = (base_pt + ci * CH) * _K
            pltpu.sync_copy(idx_hbm.at[pl.ds(r0, CH * _K)], idx_v)
            pltpu.async_copy(tab_hbm.at[idx_v], rows_v, sem).wait()
            pltpu.sync_copy(rows_v, out_hbm.at[pl.ds(r0, CH * _K)])
            return carry

        lax.fori_loop(0, n_chunks, chunk_body, 0)

    return gather


def _edge_d_body(g_ref, xi_ref, wf_ref, y_ref, mx_ref):
    C = wf_ref.shape[0] // 2
    Cout = wf_ref.shape[1]
    g = g_ref[...][:, :C]
    xi = xi_ref[...][:, :C]
    xi_exp = jnp.broadcast_to(xi[:, None, :], (_RD, _K, C)).reshape(
        _RD * _K, C)
    feat = jnp.concatenate([g - xi_exp, xi_exp], axis=1)
    y = jnp.dot(feat, wf_ref[...], preferred_element_type=jnp.float32)
    y3 = y.reshape(_RD, _K, Cout)
    mx_ref[...] = jnp.max(y3, axis=1)
    y_ref[0] = jnp.transpose(y, (1, 0)).reshape(Cout, _RD, _K)


def _edge_d(gat, xtab, WfT):
    C2, Cout = WfT.shape
    TW = xtab.shape[1]
    nb = _N // _RD
    grid = (_P // _RD,)
    y, mx = pl.pallas_call(
        _edge_d_body,
        grid=grid,
        in_specs=[
            pl.BlockSpec((_RD * _K, TW), lambda i: (i, 0)),
            pl.BlockSpec((_RD, TW), lambda i: (i, 0)),
            pl.BlockSpec((C2, Cout), lambda i: (0, 0)),
        ],
        out_specs=[
            pl.BlockSpec((1, Cout, _RD, _K),
                         lambda i: (i // nb, 0, i % nb, 0)),
            pl.BlockSpec((_RD, Cout), lambda i: (i, 0)),
        ],
        out_shape=[
            jax.ShapeDtypeStruct((_B, Cout, _N, _K), jnp.float32),
            jax.ShapeDtypeStruct((_P, Cout), jnp.float32),
        ],
    )(gat, xtab, WfT)
    return y, mx


def _edge_e_body(mx_ref, m_ref, v_ref, g_ref, b_ref, o_ref):
    out = ((mx_ref[...] - m_ref[0][None, :])
           / jnp.sqrt(v_ref[0] + _EPS)[None, :]
           * g_ref[0][None, :] + b_ref[0][None, :])
    out = jnp.maximum(out, 0.0)
    tw = o_ref.shape[1]
    if tw > out.shape[1]:
        out = jnp.concatenate(
            [out, jnp.zeros((out.shape[0], tw - out.shape[1]), jnp.float32)],
            axis=1)
    o_ref[...] = out


def _edge_e(mx, m, v, g, b, tw_out):
    P, Cout = mx.shape
    return pl.pallas_call(
        _edge_e_body,
        out_shape=jax.ShapeDtypeStruct((P, tw_out), jnp.float32),
    )(mx, m[None, :], v[None, :], g[None, :], b[None, :])


def _edge_layer(xtab, C, W, g, b):
    Cout = W.shape[0]
    WfT = jnp.transpose(W)
    TW = xtab.shape[1]
    x3d = xtab.reshape(_B, _N, TW)
    xt = jnp.transpose(x3d[:, :, :C], (0, 2, 1))
    xx = jnp.sum(xt * xt, axis=1)[:, None, :]
    idx = _knn(x3d, xx)
    gat = _sc_gather(TW)(idx.reshape(_P * _K), xtab)
    y, mx = _edge_d(gat, xtab, WfT)
    m = jnp.mean(y, axis=(0, 2, 3))
    v = jnp.var(y, axis=(0, 2, 3))
    return _edge_e(mx, m, v, g, b, max(Cout, 128))


def _head_body(x1_ref, x2_ref, x3_ref, x4_ref, w5_ref, g5_ref, b5_ref,
               w6a_ref, w6b_ref, g6_ref, b6_ref, w7_ref, g7_ref, b7_ref,
               w8_ref, b8_ref, o_ref):
    xc = jnp.concatenate(
        [x1_ref[...], x2_ref[...], x3_ref[...], x4_ref[...]], axis=1)

    sum5 = jnp.zeros((w5_ref.shape[1],), jnp.float32)
    sq5 = jnp.zeros((w5_ref.shape[1],), jnp.float32)
    m5 = []
    for bb in range(_B):
        y5b = jnp.dot(xc[bb * _N:(bb + 1) * _N], w5_ref[...],
                      preferred_element_type=jnp.float32)
        sum5 = sum5 + jnp.sum(y5b, axis=0)
        sq5 = sq5 + jnp.sum(y5b * y5b, axis=0)
        m5.append(jnp.max(y5b, axis=0, keepdims=True))
    mean5 = sum5 / _P
    var5 = sq5 / _P - mean5 * mean5
    xm = jnp.maximum(
        (jnp.concatenate(m5, axis=0) - mean5[None, :])
        / jnp.sqrt(var5 + _EPS)[None, :] * g5_ref[0][None, :]
        + b5_ref[0][None, :], 0.0)
    cvec = jnp.dot(xm, w6b_ref[...], preferred_element_type=jnp.float32)

    y6l = []
    for bb in range(_B):
        y6b = jnp.dot(xc[bb * _N:(bb + 1) * _N], w6a_ref[...],
                      preferred_element_type=jnp.float32)
        y6l.append(y6b + cvec[bb][None, :])
    y6 = jnp.concatenate(y6l, axis=0)
    mean6 = jnp.sum(y6, axis=0) / _P
    var6 = jnp.sum(y6 * y6, axis=0) / _P - mean6 * mean6
    xf6 = jnp.maximum(
        (y6 - mean6[None, :]) / jnp.sqrt(var6 + _EPS)[None, :]
        * g6_ref[0][None, :] + b6_ref[0][None, :], 0.0)

    y7 = jnp.dot(xf6, w7_ref[...], preferred_element_type=jnp.float32)
    mean7 = jnp.sum(y7, axis=0) / _P
    var7 = jnp.sum(y7 * y7, axis=0) / _P - mean7 * mean7
    xf7 = jnp.maximum(
        (y7 - mean7[None, :]) / jnp.sqrt(var7 + _EPS)[None, :]
        * g7_ref[0][None, :] + b7_ref[0][None, :], 0.0)

    y8 = jnp.dot(xf7, w8_ref[...], preferred_element_type=jnp.float32)
    o_ref[...] = y8 + b8_ref[0][None, :]


def _head(x1, x2, x3, x4, W5, g5, b5, W6, g6, b6, W7, g7, b7, W8, b8):
    W5T = jnp.transpose(W5)
    W6aT = jnp.transpose(W6[:, :512])
    W6bT = jnp.transpose(W6[:, 512:])
    W7T = jnp.transpose(W7)
    O = W8.shape[0]
    W8Tp = jnp.zeros((W8.shape[1], 128), jnp.float32).at[:, :O].set(
        jnp.transpose(W8))
    b8p = jnp.zeros((128,), jnp.float32).at[:O].set(b8)
    out = pl.pallas_call(
        _head_body,
        out_shape=jax.ShapeDtypeStruct((_P, 128), jnp.float32),
    )(x1, x2, x3, x4, W5T, g5[None, :], b5[None, :], W6aT, W6bT,
      g6[None, :], b6[None, :], W7T, g7[None, :], b7[None, :],
      W8Tp, b8p[None, :])
    return out[:, :O].reshape(_B, _N, O)


def kernel(x, W1, g1, b1, W2, g2, b2, W3, g3, b3, W4, g4, b4, W5, g5, b5,
           W6, g6, b6, W7, g7, b7, W8, b8):
    x0 = jnp.pad(x.reshape(_P, 4), ((0, 0), (0, 124)))
    t1 = _edge_layer(x0, 4, W1, g1, b1)
    t2 = _edge_layer(t1, 64, W2, g2, b2)
    t3 = _edge_layer(t2, 64, W3, g3, b3)
    t4 = _edge_layer(t3, 128, W4, g4, b4)
    return _head(t1[:, :64], t2[:, :64], t3, t4, W5, g5, b5, W6, g6, b6,
                 W7, g7, b7, W8, b8)
